# baseline (device time: 21893 ns/iter reference)
import jax
import jax.numpy as jnp
from jax import lax
from jax.experimental import pallas as pl
from jax.experimental.pallas import tpu as pltpu


def kernel(A, B):
    m, k = A.shape
    _, n = B.shape

    T = 16
    mc = m // T

    def body(
        a_ref,
        b_ref,
        out_ref,
        send_q,
        recv_q,
        send_scale,
        recv_scale,
        send_sems,
        recv_sems,
        scale_send_sem,
        scale_recv_sem,
    ):
        my_x = lax.axis_index("x")
        my_y = lax.axis_index("y")
        peer = (my_x, 1 - my_y)

        barrier_sem = pltpu.get_barrier_semaphore()
        pl.semaphore_signal(
            barrier_sem, inc=1, device_id=peer,
            device_id_type=pl.DeviceIdType.MESH,
        )
        pl.semaphore_wait(barrier_sem, 1)

        def chunk_rdma(t):
            rows = pl.ds(t * mc, mc)
            return pltpu.make_async_remote_copy(
                src_ref=send_q.at[rows, :],
                dst_ref=recv_q.at[rows, :],
                send_sem=send_sems.at[t],
                recv_sem=recv_sems.at[t],
                device_id=peer,
                device_id_type=pl.DeviceIdType.MESH,
            )

        def scale_rdma():
            return pltpu.make_async_remote_copy(
                src_ref=send_scale,
                dst_ref=recv_scale,
                send_sem=scale_send_sem,
                recv_sem=scale_recv_sem,
                device_id=peer,
                device_id_type=pl.DeviceIdType.MESH,
            )

        for t in range(T):
            rows = pl.ds(t * mc, mc)
            part = jnp.dot(
                a_ref[rows, :], b_ref[:, :],
                preferred_element_type=jnp.float32,
            )
            out_ref[rows, :] = part
            m_abs = jnp.maximum(jnp.max(jnp.abs(part)), 1e-30)
            send_scale[t, :] = jnp.full((128,), m_abs / 127.0, jnp.float32)
            send_q[rows, :] = jnp.rint(part * (127.0 / m_abs)).astype(
                jnp.int8
            )
            chunk_rdma(t).start()

        scale_rdma().start()

        scale_rdma().wait()
        for t in range(T):
            rows = pl.ds(t * mc, mc)
            chunk_rdma(t).wait()
            s = recv_scale[t : t + 1, 0:1]
            out_ref[rows, :] = (
                out_ref[rows, :] + recv_q[rows, :].astype(jnp.float32) * s
            )

    return pl.pallas_call(
        body,
        out_shape=jax.ShapeDtypeStruct((m, n), jnp.float32),
        in_specs=[
            pl.BlockSpec(memory_space=pltpu.VMEM),
            pl.BlockSpec(memory_space=pltpu.VMEM),
        ],
        out_specs=pl.BlockSpec(memory_space=pltpu.VMEM),
        scratch_shapes=[
            pltpu.VMEM((m, n), jnp.int8),
            pltpu.VMEM((m, n), jnp.int8),
            pltpu.VMEM((T, 128), jnp.float32),
            pltpu.VMEM((T, 128), jnp.float32),
            pltpu.SemaphoreType.DMA((T,)),
            pltpu.SemaphoreType.DMA((T,)),
            pltpu.SemaphoreType.DMA,
            pltpu.SemaphoreType.DMA,
        ],
        compiler_params=pltpu.CompilerParams(collective_id=0),
    )(A, B)


# device time: 21123 ns/iter; 1.0365x vs baseline; 1.0365x over previous
import jax
import jax.numpy as jnp
from jax import lax
from jax.experimental import pallas as pl
from jax.experimental.pallas import tpu as pltpu


def kernel(A, B):
    m, k = A.shape
    _, n = B.shape

    T = 16
    mc = m // T

    def body(
        a_ref,
        b_ref,
        out_ref,
        send_q,
        recv_q,
        send_scale,
        recv_scale,
        send_sems,
        recv_sems,
        scale_send_sems,
        scale_recv_sems,
    ):
        my_x = lax.axis_index("x")
        my_y = lax.axis_index("y")
        peer = (my_x, 1 - my_y)

        barrier_sem = pltpu.get_barrier_semaphore()
        pl.semaphore_signal(
            barrier_sem, inc=1, device_id=peer,
            device_id_type=pl.DeviceIdType.MESH,
        )
        pl.semaphore_wait(barrier_sem, 1)

        def chunk_rdma(t):
            rows = pl.ds(t * mc, mc)
            return pltpu.make_async_remote_copy(
                src_ref=send_q.at[rows, :],
                dst_ref=recv_q.at[rows, :],
                send_sem=send_sems.at[t],
                recv_sem=recv_sems.at[t],
                device_id=peer,
                device_id_type=pl.DeviceIdType.MESH,
            )

        def scale_rdma(t):
            return pltpu.make_async_remote_copy(
                src_ref=send_scale.at[t],
                dst_ref=recv_scale.at[t],
                send_sem=scale_send_sems.at[t],
                recv_sem=scale_recv_sems.at[t],
                device_id=peer,
                device_id_type=pl.DeviceIdType.MESH,
            )

        for t in range(T):
            rows = pl.ds(t * mc, mc)
            part = jnp.dot(
                a_ref[rows, :], b_ref[:, :],
                preferred_element_type=jnp.float32,
            )
            out_ref[rows, :] = part
            m_abs = jnp.maximum(jnp.max(jnp.abs(part)), 1e-30)
            send_scale[t, :] = jnp.full((128,), m_abs / 127.0, jnp.float32)
            send_q[rows, :] = jnp.rint(part * (127.0 / m_abs)).astype(
                jnp.int8
            )
            scale_rdma(t).start()
            chunk_rdma(t).start()

        for t in range(T):
            rows = pl.ds(t * mc, mc)
            scale_rdma(t).wait()
            chunk_rdma(t).wait()
            s = recv_scale[t : t + 1, 0:1]
            out_ref[rows, :] = (
                out_ref[rows, :] + recv_q[rows, :].astype(jnp.float32) * s
            )

    return pl.pallas_call(
        body,
        out_shape=jax.ShapeDtypeStruct((m, n), jnp.float32),
        in_specs=[
            pl.BlockSpec(memory_space=pltpu.VMEM),
            pl.BlockSpec(memory_space=pltpu.VMEM),
        ],
        out_specs=pl.BlockSpec(memory_space=pltpu.VMEM),
        scratch_shapes=[
            pltpu.VMEM((m, n), jnp.int8),
            pltpu.VMEM((m, n), jnp.int8),
            pltpu.VMEM((T, 128), jnp.float32),
            pltpu.VMEM((T, 128), jnp.float32),
            pltpu.SemaphoreType.DMA((T,)),
            pltpu.SemaphoreType.DMA((T,)),
            pltpu.SemaphoreType.DMA((T,)),
            pltpu.SemaphoreType.DMA((T,)),
        ],
        compiler_params=pltpu.CompilerParams(collective_id=0),
    )(A, B)


# device time: 20952 ns/iter; 1.0449x vs baseline; 1.0082x over previous
import jax
import jax.numpy as jnp
from jax import lax
from jax.experimental import pallas as pl
from jax.experimental.pallas import tpu as pltpu


def kernel(A, B):
    m, k = A.shape
    _, n = B.shape

    CHUNKS = [(0, 64), (64, 192), (256, 256), (512, 256), (768, 256)]
    T = len(CHUNKS)

    def body(
        a_ref,
        b_ref,
        out_ref,
        send_q,
        recv_q,
        send_scale,
        recv_scale,
        send_sems,
        recv_sems,
        scale_send_sems,
        scale_recv_sems,
    ):
        my_x = lax.axis_index("x")
        my_y = lax.axis_index("y")
        peer = (my_x, 1 - my_y)

        def chunk_rdma(t):
            off, sz = CHUNKS[t]
            rows = pl.ds(off, sz)
            return pltpu.make_async_remote_copy(
                src_ref=send_q.at[rows, :],
                dst_ref=recv_q.at[rows, :],
                send_sem=send_sems.at[t],
                recv_sem=recv_sems.at[t],
                device_id=peer,
                device_id_type=pl.DeviceIdType.MESH,
            )

        def scale_rdma(t):
            return pltpu.make_async_remote_copy(
                src_ref=send_scale.at[t],
                dst_ref=recv_scale.at[t],
                send_sem=scale_send_sems.at[t],
                recv_sem=scale_recv_sems.at[t],
                device_id=peer,
                device_id_type=pl.DeviceIdType.MESH,
            )

        def compute_chunk(t):
            off, sz = CHUNKS[t]
            rows = pl.ds(off, sz)
            part = jnp.dot(
                a_ref[rows, :], b_ref[:, :],
                preferred_element_type=jnp.float32,
            )
            out_ref[rows, :] = part
            m_abs = jnp.maximum(jnp.max(jnp.abs(part)), 1e-30)
            send_scale[t, :] = jnp.full((128,), m_abs / 127.0, jnp.float32)
            send_q[rows, :] = jnp.rint(part * (127.0 / m_abs)).astype(
                jnp.int8
            )

        barrier_sem = pltpu.get_barrier_semaphore()
        pl.semaphore_signal(
            barrier_sem, inc=1, device_id=peer,
            device_id_type=pl.DeviceIdType.MESH,
        )
        compute_chunk(0)
        pl.semaphore_wait(barrier_sem, 1)

        for t in range(T):
            if t > 0:
                compute_chunk(t)
            scale_rdma(t).start()
            chunk_rdma(t).start()

        for t in range(T):
            off, sz = CHUNKS[t]
            rows = pl.ds(off, sz)
            scale_rdma(t).wait()
            chunk_rdma(t).wait()
            s = recv_scale[t : t + 1, 0:1]
            out_ref[rows, :] = (
                out_ref[rows, :] + recv_q[rows, :].astype(jnp.float32) * s
            )

    return pl.pallas_call(
        body,
        out_shape=jax.ShapeDtypeStruct((m, n), jnp.float32),
        in_specs=[
            pl.BlockSpec(memory_space=pltpu.VMEM),
            pl.BlockSpec(memory_space=pltpu.VMEM),
        ],
        out_specs=pl.BlockSpec(memory_space=pltpu.VMEM),
        scratch_shapes=[
            pltpu.VMEM((m, n), jnp.int8),
            pltpu.VMEM((m, n), jnp.int8),
            pltpu.VMEM((T, 128), jnp.float32),
            pltpu.VMEM((T, 128), jnp.float32),
            pltpu.SemaphoreType.DMA((T,)),
            pltpu.SemaphoreType.DMA((T,)),
            pltpu.SemaphoreType.DMA((T,)),
            pltpu.SemaphoreType.DMA((T,)),
        ],
        compiler_params=pltpu.CompilerParams(collective_id=0),
    )(A, B)


# device time: 20842 ns/iter; 1.0504x vs baseline; 1.0053x over previous
import jax
import jax.numpy as jnp
from jax import lax
from jax.experimental import pallas as pl
from jax.experimental.pallas import tpu as pltpu


def kernel(A, B):
    m, k = A.shape
    _, n = B.shape

    CHUNKS = [(0, 64), (64, 192), (256, 256), (512, 256), (768, 192), (960, 64)]
    T = len(CHUNKS)

    def body(
        a_ref,
        b_ref,
        out_ref,
        acc,
        out_sems,
        send_q,
        recv_q,
        send_scale,
        recv_scale,
        send_sems,
        recv_sems,
        scale_send_sems,
        scale_recv_sems,
    ):
        my_x = lax.axis_index("x")
        my_y = lax.axis_index("y")
        peer = (my_x, 1 - my_y)

        def chunk_rdma(t):
            off, sz = CHUNKS[t]
            rows = pl.ds(off, sz)
            return pltpu.make_async_remote_copy(
                src_ref=send_q.at[rows, :],
                dst_ref=recv_q.at[rows, :],
                send_sem=send_sems.at[t],
                recv_sem=recv_sems.at[t],
                device_id=peer,
                device_id_type=pl.DeviceIdType.MESH,
            )

        def scale_rdma(t):
            return pltpu.make_async_remote_copy(
                src_ref=send_scale.at[t],
                dst_ref=recv_scale.at[t],
                send_sem=scale_send_sems.at[t],
                recv_sem=scale_recv_sems.at[t],
                device_id=peer,
                device_id_type=pl.DeviceIdType.MESH,
            )

        def compute_chunk(t):
            off, sz = CHUNKS[t]
            rows = pl.ds(off, sz)
            part = jnp.dot(
                a_ref[rows, :], b_ref[:, :],
                preferred_element_type=jnp.float32,
            )
            acc[rows, :] = part
            m_abs = jnp.maximum(jnp.max(jnp.abs(part)), 1e-30)
            send_scale[t, :] = jnp.full((128,), m_abs / 127.0, jnp.float32)
            send_q[rows, :] = jnp.rint(part * (127.0 / m_abs)).astype(
                jnp.int8
            )

        barrier_sem = pltpu.get_barrier_semaphore()
        pl.semaphore_signal(
            barrier_sem, inc=1, device_id=peer,
            device_id_type=pl.DeviceIdType.MESH,
        )
        compute_chunk(0)
        pl.semaphore_wait(barrier_sem, 1)

        for t in range(T):
            if t > 0:
                compute_chunk(t)
            scale_rdma(t).start()
            chunk_rdma(t).start()

        for t in range(T):
            off, sz = CHUNKS[t]
            rows = pl.ds(off, sz)
            scale_rdma(t).wait()
            chunk_rdma(t).wait()
            s = recv_scale[t : t + 1, 0:1]
            acc[rows, :] = (
                acc[rows, :] + recv_q[rows, :].astype(jnp.float32) * s
            )
            pltpu.make_async_copy(
                acc.at[rows, :], out_ref.at[rows, :], out_sems.at[t]
            ).start()

        for t in range(T):
            off, sz = CHUNKS[t]
            rows = pl.ds(off, sz)
            pltpu.make_async_copy(
                acc.at[rows, :], out_ref.at[rows, :], out_sems.at[t]
            ).wait()

    return pl.pallas_call(
        body,
        out_shape=jax.ShapeDtypeStruct((m, n), jnp.float32),
        in_specs=[
            pl.BlockSpec(memory_space=pltpu.VMEM),
            pl.BlockSpec(memory_space=pltpu.VMEM),
        ],
        out_specs=pl.BlockSpec(memory_space=pltpu.MemorySpace.HBM),
        scratch_shapes=[
            pltpu.VMEM((m, n), jnp.float32),
            pltpu.SemaphoreType.DMA((T,)),
            pltpu.VMEM((m, n), jnp.int8),
            pltpu.VMEM((m, n), jnp.int8),
            pltpu.VMEM((T, 128), jnp.float32),
            pltpu.VMEM((T, 128), jnp.float32),
            pltpu.SemaphoreType.DMA((T,)),
            pltpu.SemaphoreType.DMA((T,)),
            pltpu.SemaphoreType.DMA((T,)),
            pltpu.SemaphoreType.DMA((T,)),
        ],
        compiler_params=pltpu.CompilerParams(collective_id=0),
    )(A, B)


# device time: 20722 ns/iter; 1.0565x vs baseline; 1.0058x over previous
import jax
import jax.numpy as jnp
from jax import lax
from jax.experimental import pallas as pl
from jax.experimental.pallas import tpu as pltpu


def kernel(A, B):
    m, k = A.shape
    _, n = B.shape

    CHUNKS = [(0, 64), (64, 192), (256, 256), (512, 256), (768, 192), (960, 64)]
    T = len(CHUNKS)

    def body(
        a_ref,
        b_ref,
        out_ref,
        send_q,
        recv_q,
        send_scale,
        recv_scale,
        send_sems,
        recv_sems,
        scale_send_sems,
        scale_recv_sems,
    ):
        my_x = lax.axis_index("x")
        my_y = lax.axis_index("y")
        peer = (my_x, 1 - my_y)

        def chunk_rdma(t):
            off, sz = CHUNKS[t]
            rows = pl.ds(off, sz)
            return pltpu.make_async_remote_copy(
                src_ref=send_q.at[rows, :],
                dst_ref=recv_q.at[rows, :],
                send_sem=send_sems.at[t],
                recv_sem=recv_sems.at[t],
                device_id=peer,
                device_id_type=pl.DeviceIdType.MESH,
            )

        def scale_rdma(t):
            return pltpu.make_async_remote_copy(
                src_ref=send_scale.at[t],
                dst_ref=recv_scale.at[t],
                send_sem=scale_send_sems.at[t],
                recv_sem=scale_recv_sems.at[t],
                device_id=peer,
                device_id_type=pl.DeviceIdType.MESH,
            )

        def compute_chunk(t):
            off, sz = CHUNKS[t]
            rows = pl.ds(off, sz)
            part = jnp.dot(
                a_ref[rows, :], b_ref[:, :],
                preferred_element_type=jnp.float32,
            )
            out_ref[rows, :] = part
            m_abs = jnp.maximum(jnp.max(jnp.abs(part)), 1e-30)
            send_scale[t, :] = jnp.full((128,), m_abs / 127.0, jnp.float32)
            send_q[rows, :] = jnp.rint(part * (127.0 / m_abs)).astype(
                jnp.int8
            )

        barrier_sem = pltpu.get_barrier_semaphore()
        pl.semaphore_signal(
            barrier_sem, inc=1, device_id=peer,
            device_id_type=pl.DeviceIdType.MESH,
        )
        compute_chunk(0)
        pl.semaphore_wait(barrier_sem, 1)

        for t in range(T):
            if t > 0:
                compute_chunk(t)
            scale_rdma(t).start()
            chunk_rdma(t).start()

        for t in range(T):
            off, sz = CHUNKS[t]
            rows = pl.ds(off, sz)
            scale_rdma(t).wait()
            chunk_rdma(t).wait()
            s = recv_scale[t : t + 1, 0:1]
            out_ref[rows, :] = (
                out_ref[rows, :] + recv_q[rows, :].astype(jnp.float32) * s
            )

    return pl.pallas_call(
        body,
        out_shape=jax.ShapeDtypeStruct((m, n), jnp.float32),
        in_specs=[
            pl.BlockSpec(memory_space=pltpu.VMEM),
            pl.BlockSpec(memory_space=pltpu.VMEM),
        ],
        out_specs=pl.BlockSpec(memory_space=pltpu.VMEM),
        scratch_shapes=[
            pltpu.VMEM((m, n), jnp.int8),
            pltpu.VMEM((m, n), jnp.int8),
            pltpu.VMEM((T, 128), jnp.float32),
            pltpu.VMEM((T, 128), jnp.float32),
            pltpu.SemaphoreType.DMA((T,)),
            pltpu.SemaphoreType.DMA((T,)),
            pltpu.SemaphoreType.DMA((T,)),
            pltpu.SemaphoreType.DMA((T,)),
        ],
        compiler_params=pltpu.CompilerParams(collective_id=0),
    )(A, B)
